# baseline (speedup 1.0000x reference)
import functools

import jax
import jax.numpy as jnp
from jax import lax
from jax.experimental import pallas as pl
from jax.experimental.pallas import tpu as pltpu
from jax.experimental.pallas import tpu_sc as plsc

_T = 16384
_E = 64
_K = 8
_NC = 2
_NS = 16
_NW = _NC * _NS

_S = 12288
_TPW = _S // _NW
_B = 1024


def _sc_body(logits_hbm, out_w_hbm, out_i_hbm, vals_v, w_v, i_v):
  wid = lax.axis_index("s") * _NC + lax.axis_index("c")
  base = wid * _TPW
  pltpu.sync_copy(logits_hbm.at[pl.ds(base, _TPW)], vals_v)

  lanes = lax.iota(jnp.int32, 16)
  m8 = lanes < _K

  def comb(ak, av, bk, bv):
    rbk = lax.rev(bk, (0,))
    rbv = lax.rev(bv, (0,))
    take_a = ak >= rbk
    return jnp.maximum(ak, rbk), jnp.where(take_a, av, rbv)

  @plsc.parallel_loop(0, _TPW, unroll=4)
  def tok(t):
    ks = []
    vs = []
    for j in range(4):
      kj = vals_v[t, pl.ds(j * 16, 16)]
      sk, sv = plsc.sort_key_val(kj, lanes + j * 16, descending=True)
      ks.append(sk)
      vs.append(sv)
    m01k, m01v = comb(ks[0], vs[0], ks[1], vs[1])
    m23k, m23v = comb(ks[2], vs[2], ks[3], vs[3])
    t01k, t01v = plsc.sort_key_val(m01k, m01v, descending=True)
    t23k, t23v = plsc.sort_key_val(m23k, m23v, descending=True)
    fk0, fv0 = comb(t01k, t01v, t23k, t23v)
    fk, fv = plsc.sort_key_val(fk0, fv0, descending=True)
    ssum = jnp.sum(jnp.where(m8, fk, 0.0))
    w = fk / ssum
    off = pl.multiple_of(t * _K, 8)
    plsc.store_compressed(w_v.at[pl.ds(off, 16)], w, mask=m8)
    plsc.store_compressed(i_v.at[pl.ds(off, 16)], fv, mask=m8)

  n = _TPW * _K
  pltpu.sync_copy(w_v.at[pl.ds(0, n)], out_w_hbm.at[pl.ds(base * _K, n)])
  pltpu.sync_copy(i_v.at[pl.ds(0, n)], out_i_hbm.at[pl.ds(base * _K, n)])


_sc_call = pl.kernel(
    _sc_body,
    out_type=(
        jax.ShapeDtypeStruct((_S * _K,), jnp.float32),
        jax.ShapeDtypeStruct((_S * _K,), jnp.int32),
    ),
    mesh=plsc.VectorSubcoreMesh(
        core_axis_name="c", subcore_axis_name="s",
        num_cores=_NC, num_subcores=_NS),
    scratch_types=[
        pltpu.VMEM((_TPW, _E), jnp.float32),
        pltpu.VMEM((_TPW * _K + 8,), jnp.float32),
        pltpu.VMEM((_TPW * _K + 8,), jnp.int32),
    ],
    compiler_params=pltpu.CompilerParams(needs_layout_passes=False),
)


def _tc_body(x_ref, w_ref, i_ref):
  x = x_ref[...]
  iota_f = lax.broadcasted_iota(jnp.int32, (_B, _E), 1).astype(jnp.float32)
  ws = []
  ids = []
  for _ in range(_K):
    m = jnp.max(x, axis=1, keepdims=True)
    eq = x == m
    first = jnp.min(jnp.where(eq, iota_f, float(_E)), axis=1, keepdims=True)
    ws.append(m)
    ids.append(first.astype(jnp.int32))
    x = jnp.where(iota_f == first, -jnp.inf, x)
  W = jnp.concatenate(ws, axis=1)
  I = jnp.concatenate(ids, axis=1)
  s = jnp.sum(W, axis=1, keepdims=True)
  w_ref[...] = W / s
  i_ref[...] = I


_tc_call = pl.pallas_call(
    _tc_body,
    grid=((_T - _S) // _B,),
    in_specs=[pl.BlockSpec((_B, _E), lambda i: (i + _S // _B, 0))],
    out_specs=[
        pl.BlockSpec((_B, _K), lambda i: (i, 0)),
        pl.BlockSpec((_B, _K), lambda i: (i, 0)),
    ],
    out_shape=[
        jax.ShapeDtypeStruct((_T - _S, _K), jnp.float32),
        jax.ShapeDtypeStruct((_T - _S, _K), jnp.int32),
    ],
)


@jax.jit
def kernel(router_logits):
  x = router_logits.astype(jnp.float32)
  w_sc, i_sc = _sc_call(x)
  w_tc, i_tc = _tc_call(x)
  w = jnp.concatenate([w_sc.reshape(_S, _K), w_tc], axis=0)
  i = jnp.concatenate([i_sc.reshape(_S, _K), i_tc], axis=0)
  return w, i


# speedup vs baseline: 1.7251x; 1.7251x over previous
import jax
import jax.numpy as jnp
from jax import lax
from jax.experimental import pallas as pl
from jax.experimental.pallas import tpu as pltpu
from jax.experimental.pallas import tpu_sc as plsc

_T = 16384
_E = 64
_K = 8
_NC = 2
_NS = 16
_NW = _NC * _NS

_TPW = _T // _NW


def _lane_take(v, idx):
  return lax.gather(
      v, idx[:, None],
      lax.GatherDimensionNumbers(
          offset_dims=(), collapsed_slice_dims=(0,), start_index_map=(0,)),
      (1,),
      mode=lax.GatherScatterMode.PROMISE_IN_BOUNDS)


def _sc_body(logits_hbm, out_w_hbm, out_i_hbm, vals_v, w_v, i_v, dsem):
  wid = lax.axis_index("s") * _NC + lax.axis_index("c")
  base = wid * _TPW

  lanes = lax.iota(jnp.int32, 16)
  m8 = lanes < _K
  p_a = lanes ^ 1
  p_b = jnp.clip(((lanes - 1) ^ 1) + 1, 0, 15)
  nchunk = 4
  cs = _TPW // nchunk

  def comb(ak, av, bk, bv):
    rbk = lax.rev(bk, (0,))
    rbv = lax.rev(bv, (0,))
    take_a = ak >= rbk
    return jnp.maximum(ak, rbk), jnp.where(take_a, av, rbv)

  cp0 = pltpu.make_async_copy(
      logits_hbm.at[pl.ds(base, cs)], vals_v.at[pl.ds(0, cs)], dsem)
  cp0.start()
  for c in range(nchunk):
    pltpu.make_async_copy(
        logits_hbm.at[pl.ds(base + c * cs, cs)],
        vals_v.at[pl.ds(c * cs, cs)], dsem).wait()
    if c + 1 < nchunk:
      pltpu.make_async_copy(
          logits_hbm.at[pl.ds(base + (c + 1) * cs, cs)],
          vals_v.at[pl.ds((c + 1) * cs, cs)], dsem).start()

    @plsc.parallel_loop(c * cs, (c + 1) * cs, unroll=4)
    def tok(t):
      ks = []
      vs = []
      for j in range(4):
        kj = vals_v[t, pl.ds(j * 16, 16)]
        sk, sv = plsc.sort_key_val(kj, lanes + j * 16, descending=True)
        ks.append(sk)
        vs.append(sv)
      m01k, m01v = comb(ks[0], vs[0], ks[1], vs[1])
      m23k, m23v = comb(ks[2], vs[2], ks[3], vs[3])
      t01k, t01v = plsc.sort_key_val(m01k, m01v, descending=True)
      t23k, t23v = plsc.sort_key_val(m23k, m23v, descending=True)
      fk0, fv0 = comb(t01k, t01v, t23k, t23v)
      fk, fv = plsc.sort_key_val(fk0, fv0, descending=True)
      for p in (p_a, p_b):
        vp = _lane_take(fk, p)
        ip = _lane_take(fv, p)
        tie = fk == vp
        pick_lo = lanes < p
        fv = jnp.where(tie, jnp.where(pick_lo, jnp.minimum(fv, ip),
                                      jnp.maximum(fv, ip)), fv)
      ssum = jnp.sum(jnp.where(m8, fk, 0.0))
      w = fk / ssum
      tv = jnp.full((16,), 0, jnp.int32) + t
      plsc.store_scatter(w_v, [lanes, tv], w, mask=m8)
      plsc.store_scatter(i_v, [lanes, tv], fv, mask=m8)

  for k in range(_K):
    pltpu.sync_copy(w_v.at[k], out_w_hbm.at[k, pl.ds(base, _TPW)])
    pltpu.sync_copy(i_v.at[k], out_i_hbm.at[k, pl.ds(base, _TPW)])


_sc_call = pl.kernel(
    _sc_body,
    out_type=(
        jax.ShapeDtypeStruct((_K, _T), jnp.float32),
        jax.ShapeDtypeStruct((_K, _T), jnp.int32),
    ),
    mesh=plsc.VectorSubcoreMesh(
        core_axis_name="c", subcore_axis_name="s",
        num_cores=_NC, num_subcores=_NS),
    scratch_types=[
        pltpu.VMEM((_TPW, _E), jnp.float32),
        pltpu.VMEM((_K, _TPW), jnp.float32),
        pltpu.VMEM((_K, _TPW), jnp.int32),
        pltpu.SemaphoreType.DMA,
    ],
    compiler_params=pltpu.CompilerParams(needs_layout_passes=False, use_tc_tiling_on_sc=True),
)


@jax.jit
def kernel(router_logits):
  x = router_logits.astype(jnp.float32)
  w_sc, i_sc = _sc_call(x)
  return w_sc.T, i_sc.T
